# baseline (speedup 1.0000x reference)
import functools

import jax
import jax.numpy as jnp
from jax import lax
from jax.experimental import pallas as pl
from jax.experimental.pallas import tpu as pltpu
from jax.experimental.pallas import tpu_sc as plsc

B = 4096
L = 50
F = 13
V = 100000
D = 16
C = 16

NC = 2
NS = 16
NW = NC * NS
BPW = B // NW
G = 128
NUBS = 2 * F
HALF = B // 2

def _tables_body(item_ref, prof_ref, wt_ref, wu_ref, wp_ref,
                 ot_ref, ou_ref, op_ref):
    a = item_ref[0]
    p = prof_ref[0]
    ot_ref[0, 0, 0] = jnp.dot(a, wt_ref[0], preferred_element_type=jnp.float32)[:, 0]
    ou_ref[0, 0, 0] = jnp.dot(a, wu_ref[0], preferred_element_type=jnp.float32)[:, 0]
    op_ref[0, 0, 0] = jnp.dot(p, wp_ref[0], preferred_element_type=jnp.float32)[:, 0]


VCH = 4
VB = V // VCH


def _build_tables(item_emb, profile_emb, wt, wu, wp):
    emb_spec = pl.BlockSpec((1, VB, D), lambda f, j: (f, j, 0))
    w_spec = pl.BlockSpec((1, D, 1), lambda f, j: (f, 0, 0))
    out_spec = pl.BlockSpec((1, 1, 1, VB), lambda f, j: (f, j, 0, 0))
    out_shape = jax.ShapeDtypeStruct((F, VCH, 1, VB), jnp.float32)
    return pl.pallas_call(
        _tables_body,
        grid=(F, VCH),
        in_specs=[emb_spec, emb_spec, w_spec, w_spec, w_spec],
        out_specs=[out_spec, out_spec, out_spec],
        out_shape=[out_shape, out_shape, out_shape],
    )(item_emb, profile_emb, wt[:, :, None], wu[:, :, None], wp[:, :, None])


def _sc_body(ubs_t, tgt_f, prof_f, tub, ttgt, tprof, out,
             table_v, gidx_v, idx_v, val_v, acc_v, sem):
    wid = lax.axis_index("s") * NC + lax.axis_index("c")
    b0 = wid * BPW

    def zero_one(i, _):
        acc_v[pl.ds(i * 16, 16)] = jnp.zeros((16,), jnp.float32)
        return 0

    lax.fori_loop(0, B // 16, zero_one, 0)

    def flat_pass(idx_src, tab_flat):
        def fbody(f, _):
            pltpu.sync_copy(idx_src.at[pl.ds(f * B + b0, BPW)], idx_v)

            def add_one(k, _):
                s = pl.ds(k * 16, 16)
                idx_v[s] = idx_v[s] + f * V
                return 0

            lax.fori_loop(0, BPW // 16, add_one, 0)
            pltpu.async_copy(tab_flat.at[idx_v], val_v, sem).wait()

            def acc_one(k, _):
                d = pl.ds(b0 + k * 16, 16)
                acc_v[d] = acc_v[d] + val_v[pl.ds(k * 16, 16)]
                return 0

            lax.fori_loop(0, BPW // 16, acc_one, 0)
            return 0

        lax.fori_loop(0, F, fbody, 0)

    flat_pass(tgt_f, ttgt)
    flat_pass(prof_f, tprof)

    @pl.when(wid < NUBS)
    def _():
        f = wid // 2
        base = (wid % 2) * HALF
        pltpu.sync_copy(tub.at[pl.ds(f * V, V)], table_v)

        def gbody(g, _):
            bb = base + g * G
            pltpu.sync_copy(ubs_t.at[f, :, pl.ds(bb, G)], gidx_v)

            def kbody(k, _):
                s16 = jnp.zeros((16,), jnp.float32)
                for l in range(L):
                    s16 = s16 + plsc.load_gather(
                        table_v, [gidx_v[l, pl.ds(k * 16, 16)]])
                d = pl.ds(bb + k * 16, 16)
                acc_v[d] = acc_v[d] + s16
                return 0

            lax.fori_loop(0, G // 16, kbody, 0)
            return 0

        lax.fori_loop(0, HALF // G, gbody, 0)

    pltpu.sync_copy(acc_v, out.at[pl.ds(wid * B, B)])


@functools.cache
def _sc_gather_fn():
    return functools.partial(
        pl.kernel,
        out_type=jax.ShapeDtypeStruct((NW * B,), jnp.float32),
        mesh=plsc.VectorSubcoreMesh(core_axis_name="c", subcore_axis_name="s",
                                    num_cores=NC, num_subcores=NS),
        scratch_types=[
            pltpu.VMEM((V,), jnp.float32),
            pltpu.VMEM((L, G), jnp.int32),
            pltpu.VMEM((BPW,), jnp.int32),
            pltpu.VMEM((BPW,), jnp.float32),
            pltpu.VMEM((B,), jnp.float32),
            pltpu.SemaphoreType.DMA,
        ],
        compiler_params=pltpu.CompilerParams(needs_layout_passes=False),
    )(_sc_body)


def _head_body(p_ref, ctx_ref, wc_ref, b_ref, o_ref):
    s = jnp.sum(p_ref[...], axis=0)
    c = jnp.dot(ctx_ref[...], wc_ref[...], preferred_element_type=jnp.float32)
    logit = s[:, None] + c + b_ref[0, 0]
    o_ref[...] = jax.nn.sigmoid(logit)


def _head(partials, context, wc, bias):
    return pl.pallas_call(
        _head_body,
        out_shape=jax.ShapeDtypeStruct((B, 1), jnp.float32),
    )(partials, context, wc, bias)


def kernel(target_ad, ubs_feature, profile_feature, context_feature,
           item_emb, profile_emb, W, b):
    wt = W[:F * D, 0].reshape(F, D)
    wu = W[F * D:2 * F * D, 0].reshape(F, D) / L
    wp = W[2 * F * D:3 * F * D, 0].reshape(F, D)
    wc = W[3 * F * D:, :]

    t_tgt, t_ubs, t_prof = _build_tables(item_emb, profile_emb, wt, wu, wp)

    tub = t_ubs.reshape(F * V)
    ttgt = t_tgt.reshape(F * V)
    tprof = t_prof.reshape(F * V)

    ubs_t = jnp.transpose(ubs_feature, (2, 1, 0))
    tgt_f = target_ad.T.reshape(F * B)
    prof_f = profile_feature.T.reshape(F * B)

    partials = _sc_gather_fn()(ubs_t, tgt_f, prof_f, tub, ttgt, tprof)

    return _head(partials.reshape(NW, B), context_feature, wc, b.reshape(1, 1))


# speedup vs baseline: 1.8311x; 1.8311x over previous
import functools

import jax
import jax.numpy as jnp
from jax import lax
from jax.experimental import pallas as pl
from jax.experimental.pallas import tpu as pltpu
from jax.experimental.pallas import tpu_sc as plsc

B = 4096
L = 50
F = 13
V = 100000
D = 16
C = 16

NC = 2
NS = 16
NW = NC * NS
BPW = B // NW


def _sc_body(ubs_t, tgt_f, prof_f, item2d, prof2d, wcat, out,
             w_v, gidx_v, idx_a, idx_b, rows_a, rows_b, racc_v, acc_v,
             sem_a, sem_b):
    wid = lax.axis_index("s") * NC + lax.axis_index("c")
    b0 = wid * BPW

    pltpu.sync_copy(wcat, w_v)

    def zero16(i, _):
        racc_v[i] = jnp.zeros((D,), jnp.float32)
        return 0

    lax.fori_loop(0, BPW, zero16, 0)

    def accumulate(rows_v, wf):
        def acc8(i, _):
            for j in range(8):
                plsc.addupdate(racc_v.at[i * 8 + j], rows_v[i * 8 + j] * wf)
            return 0

        lax.fori_loop(0, BPW // 8, acc8, 0)

    def build_idx(dst, row, fv):
        def add16(k, _):
            s = pl.ds(k * 16, 16)
            dst[s] = gidx_v[row, s] + fv
            return 0

        lax.fori_loop(0, BPW // 16, add16, 0)

    def fbody(f, _):
        pltpu.sync_copy(ubs_t.at[f, :, pl.ds(b0, BPW)], gidx_v)
        wf = w_v[pl.ds(F * D + f * D, D)]
        fv = f * V

        build_idx(idx_a, 0, fv)
        pltpu.async_copy(item2d.at[idx_a], rows_a, sem_a)

        def lbody(l, _):
            @pl.when(l % 2 == 0)
            def _():
                @pl.when(l + 1 < L)
                def _():
                    build_idx(idx_b, l + 1, fv)
                    pltpu.async_copy(item2d.at[idx_b], rows_b, sem_b)
                pltpu.make_async_copy(item2d.at[idx_a], rows_a, sem_a).wait()
                accumulate(rows_a, wf)

            @pl.when(l % 2 == 1)
            def _():
                @pl.when(l + 1 < L)
                def _():
                    build_idx(idx_a, l + 1, fv)
                    pltpu.async_copy(item2d.at[idx_a], rows_a, sem_a)
                pltpu.make_async_copy(item2d.at[idx_b], rows_b, sem_b).wait()
                accumulate(rows_b, wf)

            return 0

        lax.fori_loop(0, L, lbody, 0)
        return 0

    lax.fori_loop(0, F, fbody, 0)

    def flat_pass(idx_src, table, woff):
        def fbody2(f, _):
            pltpu.sync_copy(idx_src.at[pl.ds(f * B + b0, BPW)], idx_a)

            def add16(k, _):
                s = pl.ds(k * 16, 16)
                idx_a[s] = idx_a[s] + f * V
                return 0

            lax.fori_loop(0, BPW // 16, add16, 0)
            pltpu.async_copy(table.at[idx_a], rows_a, sem_a).wait()
            wf = w_v[pl.ds(woff + f * D, D)]
            accumulate(rows_a, wf)
            return 0

        lax.fori_loop(0, F, fbody2, 0)

    flat_pass(tgt_f, item2d, 0)
    flat_pass(prof_f, prof2d, 2 * F * D)

    lanes = lax.iota(jnp.int32, 16)

    def red(i, _):
        rows16 = lanes + i * 16
        s = jnp.zeros((16,), jnp.float32)
        for d in range(D):
            col = jnp.full((16,), d, jnp.int32)
            s = s + plsc.load_gather(racc_v, [rows16, col])
        acc_v[pl.ds(i * 16, 16)] = s
        return 0

    lax.fori_loop(0, BPW // 16, red, 0)
    pltpu.sync_copy(acc_v, out.at[pl.ds(b0, BPW)])


@functools.cache
def _sc_gather_fn():
    return functools.partial(
        pl.kernel,
        out_type=jax.ShapeDtypeStruct((B,), jnp.float32),
        mesh=plsc.VectorSubcoreMesh(core_axis_name="c", subcore_axis_name="s",
                                    num_cores=NC, num_subcores=NS),
        scratch_types=[
            pltpu.VMEM((3 * F * D,), jnp.float32),
            pltpu.VMEM((L, BPW), jnp.int32),
            pltpu.VMEM((BPW,), jnp.int32),
            pltpu.VMEM((BPW,), jnp.int32),
            pltpu.VMEM((BPW, D), jnp.float32),
            pltpu.VMEM((BPW, D), jnp.float32),
            pltpu.VMEM((BPW, D), jnp.float32),
            pltpu.VMEM((BPW,), jnp.float32),
            pltpu.SemaphoreType.DMA,
            pltpu.SemaphoreType.DMA,
        ],
        compiler_params=pltpu.CompilerParams(needs_layout_passes=False,
                                             use_tc_tiling_on_sc=False),
    )(_sc_body)


def _head_body(s_ref, ctx_ref, wc_ref, b_ref, o_ref):
    c = jnp.dot(ctx_ref[...], wc_ref[...], preferred_element_type=jnp.float32)
    logit = s_ref[0][:, None] + c + b_ref[0, 0]
    o_ref[...] = jax.nn.sigmoid(logit)


def _head(sums, context, wc, bias):
    return pl.pallas_call(
        _head_body,
        out_shape=jax.ShapeDtypeStruct((B, 1), jnp.float32),
    )(sums, context, wc, bias)


def kernel(target_ad, ubs_feature, profile_feature, context_feature,
           item_emb, profile_emb, W, b):
    wt = W[:F * D, 0]
    wu = W[F * D:2 * F * D, 0] / L
    wp = W[2 * F * D:3 * F * D, 0]
    wc = W[3 * F * D:, :]
    wcat = jnp.concatenate([wt, wu, wp])

    item2d = item_emb.reshape(F * V, D)
    prof2d = profile_emb.reshape(F * V, D)
    ubs_t = jnp.transpose(ubs_feature, (2, 1, 0))
    tgt_f = target_ad.T.reshape(F * B)
    prof_f = profile_feature.T.reshape(F * B)

    sums = _sc_gather_fn()(ubs_t, tgt_f, prof_f, item2d, prof2d, wcat)

    return _head(sums.reshape(1, B), context_feature, wc, b.reshape(1, 1))


# speedup vs baseline: 2.1278x; 1.1621x over previous
import functools

import jax
import jax.numpy as jnp
from jax import lax
from jax.experimental import pallas as pl
from jax.experimental.pallas import tpu as pltpu
from jax.experimental.pallas import tpu_sc as plsc

B = 4096
L = 50
F = 13
V = 100000
D = 16
C = 16

NC = 2
NS = 16
NW = NC * NS
BPW = B // NW


def _zero_racc(racc_v):
    def zero16(i, _):
        racc_v[i] = jnp.zeros((D,), jnp.float32)
        return 0

    lax.fori_loop(0, BPW, zero16, 0)


def _accumulate(racc_v, rows_v, wf):
    for i in range(BPW):
        plsc.addupdate(racc_v.at[i], rows_v[i] * wf)


def _flat_gather_pass(idx_src, table, woff, b0, w_v, idx_v, rows_v, racc_v,
                      sem):
    def fbody(f, _):
        pltpu.sync_copy(idx_src.at[pl.ds(f * B + b0, BPW)], idx_v)

        def add16(k, _):
            s = pl.ds(k * 16, 16)
            idx_v[s] = idx_v[s] + f * V
            return 0

        lax.fori_loop(0, BPW // 16, add16, 0)
        pltpu.async_copy(table.at[idx_v], rows_v, sem).wait()
        wf = w_v[pl.ds(woff + f * D, D)]
        _accumulate(racc_v, rows_v, wf)
        return 0

    lax.fori_loop(0, F, fbody, 0)


def _reduce_out(racc_v, acc_v, out, b0):
    lanes = lax.iota(jnp.int32, 16)

    def red(i, _):
        rows16 = lanes + i * 16
        s = jnp.zeros((16,), jnp.float32)
        for d in range(D):
            col = jnp.full((16,), d, jnp.int32)
            s = s + plsc.load_gather(racc_v, [rows16, col])
        acc_v[pl.ds(i * 16, 16)] = s
        return 0

    lax.fori_loop(0, BPW // 16, red, 0)
    pltpu.sync_copy(acc_v, out.at[pl.ds(b0, BPW)])


def _sc_main_body(ubs_t, tgt_f, item2d, wcat, out,
                  w_v, gidx_v, idx_v, rows_a, rows_b, racc_v, acc_v,
                  sem_a, sem_b):
    wid = lax.axis_index("s") * NC + lax.axis_index("c")
    b0 = wid * BPW

    pltpu.sync_copy(wcat, w_v)
    _zero_racc(racc_v)

    def fbody(f, _):
        pltpu.sync_copy(ubs_t.at[f, :, pl.ds(b0, BPW)], gidx_v)
        wf = w_v[pl.ds(F * D + f * D, D)]
        fv = f * V

        def bias_row(l, _):
            def add16(k, _):
                s = pl.ds(k * 16, 16)
                gidx_v[l, s] = gidx_v[l, s] + fv
                return 0

            lax.fori_loop(0, BPW // 16, add16, 0)
            return 0

        lax.fori_loop(0, L, bias_row, 0)

        pltpu.async_copy(item2d.at[gidx_v.at[0]], rows_a, sem_a)

        def lbody(l, _):
            @pl.when(l % 2 == 0)
            def _():
                @pl.when(l + 1 < L)
                def _():
                    pltpu.async_copy(item2d.at[gidx_v.at[l + 1]], rows_b,
                                     sem_b)
                pltpu.make_async_copy(item2d.at[gidx_v.at[l]], rows_a,
                                      sem_a).wait()
                _accumulate(racc_v, rows_a, wf)

            @pl.when(l % 2 == 1)
            def _():
                @pl.when(l + 1 < L)
                def _():
                    pltpu.async_copy(item2d.at[gidx_v.at[l + 1]], rows_a,
                                     sem_a)
                pltpu.make_async_copy(item2d.at[gidx_v.at[l]], rows_b,
                                      sem_b).wait()
                _accumulate(racc_v, rows_b, wf)

            return 0

        lax.fori_loop(0, L, lbody, 0)
        return 0

    lax.fori_loop(0, F, fbody, 0)

    _flat_gather_pass(tgt_f, item2d, 0, b0, w_v, idx_v, rows_a, racc_v,
                      sem_a)

    _reduce_out(racc_v, acc_v, out, b0)


def _sc_prof_body(prof_f, prof2d, wcat, out,
                  w_v, idx_v, rows_a, racc_v, acc_v, sem_a):
    wid = lax.axis_index("s") * NC + lax.axis_index("c")
    b0 = wid * BPW

    pltpu.sync_copy(wcat, w_v)
    _zero_racc(racc_v)
    _flat_gather_pass(prof_f, prof2d, 2 * F * D, b0, w_v, idx_v, rows_a,
                      racc_v, sem_a)
    _reduce_out(racc_v, acc_v, out, b0)


_SC_PARAMS = dict(
    out_type=jax.ShapeDtypeStruct((B,), jnp.float32),
    compiler_params=pltpu.CompilerParams(needs_layout_passes=False,
                                         use_tc_tiling_on_sc=False),
)


@functools.cache
def _sc_fns():
    mesh = plsc.VectorSubcoreMesh(core_axis_name="c", subcore_axis_name="s",
                                  num_cores=NC, num_subcores=NS)
    main = functools.partial(
        pl.kernel,
        mesh=mesh,
        scratch_types=[
            pltpu.VMEM((3 * F * D,), jnp.float32),
            pltpu.VMEM((L, BPW), jnp.int32),
            pltpu.VMEM((BPW,), jnp.int32),
            pltpu.VMEM((BPW, D), jnp.float32),
            pltpu.VMEM((BPW, D), jnp.float32),
            pltpu.VMEM((BPW, D), jnp.float32),
            pltpu.VMEM((BPW,), jnp.float32),
            pltpu.SemaphoreType.DMA,
            pltpu.SemaphoreType.DMA,
        ],
        **_SC_PARAMS,
    )(_sc_main_body)
    prof = functools.partial(
        pl.kernel,
        mesh=mesh,
        scratch_types=[
            pltpu.VMEM((3 * F * D,), jnp.float32),
            pltpu.VMEM((BPW,), jnp.int32),
            pltpu.VMEM((BPW, D), jnp.float32),
            pltpu.VMEM((BPW, D), jnp.float32),
            pltpu.VMEM((BPW,), jnp.float32),
            pltpu.SemaphoreType.DMA,
        ],
        **_SC_PARAMS,
    )(_sc_prof_body)
    return main, prof


def _head_body(s1_ref, s2_ref, ctx_ref, wc_ref, b_ref, o_ref):
    c = jnp.dot(ctx_ref[...], wc_ref[...], preferred_element_type=jnp.float32)
    logit = (s1_ref[0] + s2_ref[0])[:, None] + c + b_ref[0, 0]
    o_ref[...] = jax.nn.sigmoid(logit)


def _head(s1, s2, context, wc, bias):
    return pl.pallas_call(
        _head_body,
        out_shape=jax.ShapeDtypeStruct((B, 1), jnp.float32),
    )(s1, s2, context, wc, bias)


def kernel(target_ad, ubs_feature, profile_feature, context_feature,
           item_emb, profile_emb, W, b):
    wt = W[:F * D, 0]
    wu = W[F * D:2 * F * D, 0] / L
    wp = W[2 * F * D:3 * F * D, 0]
    wc = W[3 * F * D:, :]
    wcat = jnp.concatenate([wt, wu, wp])

    item2d = item_emb.reshape(F * V, D)
    prof2d = profile_emb.reshape(F * V, D)
    ubs_t = jnp.transpose(ubs_feature, (2, 1, 0))
    tgt_f = target_ad.T.reshape(F * B)
    prof_f = profile_feature.T.reshape(F * B)

    main_fn, prof_fn = _sc_fns()
    s1 = main_fn(ubs_t, tgt_f, item2d, wcat)
    s2 = prof_fn(prof_f, prof2d, wcat)

    return _head(s1.reshape(1, B), s2.reshape(1, B), context_feature, wc,
                 b.reshape(1, 1))


# speedup vs baseline: 2.4170x; 1.1359x over previous
import functools

import jax
import jax.numpy as jnp
from jax import lax
from jax.experimental import pallas as pl
from jax.experimental.pallas import tpu as pltpu
from jax.experimental.pallas import tpu_sc as plsc

B = 4096
L = 50
F = 13
V = 100000
D = 16
C = 16

NC = 2
NS = 16
NW = NC * NS
BPW = B // NW


def _zero_racc(racc_v):
    def zero16(i, _):
        racc_v[i] = jnp.zeros((D,), jnp.float32)
        return 0

    lax.fori_loop(0, BPW, zero16, 0)


def _accumulate(racc_v, rows_v, wf):
    def acc8(i, _):
        for j in range(8):
            plsc.addupdate(racc_v.at[i * 8 + j], rows_v[i * 8 + j] * wf)
        return 0

    lax.fori_loop(0, BPW // 8, acc8, 0)


def _flat_gather_pass(idx_src, table, woff, b0, w_v, idx_v, rows_v, racc_v,
                      sem):
    def fbody(f, _):
        pltpu.sync_copy(idx_src.at[pl.ds(f * B + b0, BPW)], idx_v)

        def add16(k, _):
            s = pl.ds(k * 16, 16)
            idx_v[s] = idx_v[s] + f * V
            return 0

        lax.fori_loop(0, BPW // 16, add16, 0)
        pltpu.async_copy(table.at[idx_v], rows_v, sem).wait()
        wf = w_v[pl.ds(woff + f * D, D)]
        _accumulate(racc_v, rows_v, wf)
        return 0

    lax.fori_loop(0, F, fbody, 0)


def _reduce_out(racc_v, acc_v, out, b0):
    lanes = lax.iota(jnp.int32, 16)

    def red(i, _):
        rows16 = lanes + i * 16
        s = jnp.zeros((16,), jnp.float32)
        for d in range(D):
            col = jnp.full((16,), d, jnp.int32)
            s = s + plsc.load_gather(racc_v, [rows16, col])
        acc_v[pl.ds(i * 16, 16)] = s
        return 0

    lax.fori_loop(0, BPW // 16, red, 0)
    pltpu.sync_copy(acc_v, out.at[pl.ds(b0, BPW)])


def _sc_main_body(ubs_t, tgt_f, item2d, wcat, out,
                  w_v, gidx_v, idx_v, rows_a, rows_b, racc_v, acc_v,
                  sem_a, sem_b):
    wid = lax.axis_index("s") * NC + lax.axis_index("c")
    b0 = wid * BPW

    pltpu.sync_copy(wcat, w_v)
    _zero_racc(racc_v)

    def fbody(f, _):
        pltpu.sync_copy(ubs_t.at[f, :, pl.ds(b0, BPW)], gidx_v)
        wf = w_v[pl.ds(F * D + f * D, D)]
        fv = f * V

        def bias_row(l, _):
            def add16(k, _):
                s = pl.ds(k * 16, 16)
                gidx_v[l, s] = gidx_v[l, s] + fv
                return 0

            lax.fori_loop(0, BPW // 16, add16, 0)
            return 0

        lax.fori_loop(0, L, bias_row, 0)

        pltpu.async_copy(item2d.at[gidx_v.at[0]], rows_a, sem_a)

        def lbody(l, _):
            @pl.when(l % 2 == 0)
            def _():
                @pl.when(l + 1 < L)
                def _():
                    pltpu.async_copy(item2d.at[gidx_v.at[l + 1]], rows_b,
                                     sem_b)
                pltpu.make_async_copy(item2d.at[gidx_v.at[l]], rows_a,
                                      sem_a).wait()
                _accumulate(racc_v, rows_a, wf)

            @pl.when(l % 2 == 1)
            def _():
                @pl.when(l + 1 < L)
                def _():
                    pltpu.async_copy(item2d.at[gidx_v.at[l + 1]], rows_a,
                                     sem_a)
                pltpu.make_async_copy(item2d.at[gidx_v.at[l]], rows_b,
                                      sem_b).wait()
                _accumulate(racc_v, rows_b, wf)

            return 0

        lax.fori_loop(0, L, lbody, 0)
        return 0

    lax.fori_loop(0, F, fbody, 0)

    _flat_gather_pass(tgt_f, item2d, 0, b0, w_v, idx_v, rows_a, racc_v,
                      sem_a)

    _reduce_out(racc_v, acc_v, out, b0)


def _sc_prof_body(prof_f, prof2d, wcat, out,
                  w_v, idx_v, rows_a, racc_v, acc_v, sem_a):
    wid = lax.axis_index("s") * NC + lax.axis_index("c")
    b0 = wid * BPW

    pltpu.sync_copy(wcat, w_v)
    _zero_racc(racc_v)
    _flat_gather_pass(prof_f, prof2d, 2 * F * D, b0, w_v, idx_v, rows_a,
                      racc_v, sem_a)
    _reduce_out(racc_v, acc_v, out, b0)


_SC_PARAMS = dict(
    out_type=jax.ShapeDtypeStruct((B,), jnp.float32),
    compiler_params=pltpu.CompilerParams(needs_layout_passes=False,
                                         use_tc_tiling_on_sc=False),
)


@functools.cache
def _sc_fns():
    mesh = plsc.VectorSubcoreMesh(core_axis_name="c", subcore_axis_name="s",
                                  num_cores=NC, num_subcores=NS)
    main = functools.partial(
        pl.kernel,
        mesh=mesh,
        scratch_types=[
            pltpu.VMEM((3 * F * D,), jnp.float32),
            pltpu.VMEM((L, BPW), jnp.int32),
            pltpu.VMEM((BPW,), jnp.int32),
            pltpu.VMEM((BPW, D), jnp.float32),
            pltpu.VMEM((BPW, D), jnp.float32),
            pltpu.VMEM((BPW, D), jnp.float32),
            pltpu.VMEM((BPW,), jnp.float32),
            pltpu.SemaphoreType.DMA,
            pltpu.SemaphoreType.DMA,
        ],
        **_SC_PARAMS,
    )(_sc_main_body)
    prof = functools.partial(
        pl.kernel,
        mesh=mesh,
        scratch_types=[
            pltpu.VMEM((3 * F * D,), jnp.float32),
            pltpu.VMEM((BPW,), jnp.int32),
            pltpu.VMEM((BPW, D), jnp.float32),
            pltpu.VMEM((BPW, D), jnp.float32),
            pltpu.VMEM((BPW,), jnp.float32),
            pltpu.SemaphoreType.DMA,
        ],
        **_SC_PARAMS,
    )(_sc_prof_body)
    return main, prof


def _head_body(s1_ref, s2_ref, ctx_ref, wc_ref, b_ref, o_ref):
    c = jnp.dot(ctx_ref[...], wc_ref[...], preferred_element_type=jnp.float32)
    logit = (s1_ref[0] + s2_ref[0])[:, None] + c + b_ref[0, 0]
    o_ref[...] = jax.nn.sigmoid(logit)


def _head(s1, s2, context, wc, bias):
    return pl.pallas_call(
        _head_body,
        out_shape=jax.ShapeDtypeStruct((B, 1), jnp.float32),
    )(s1, s2, context, wc, bias)


def kernel(target_ad, ubs_feature, profile_feature, context_feature,
           item_emb, profile_emb, W, b):
    wt = W[:F * D, 0]
    wu = W[F * D:2 * F * D, 0] / L
    wp = W[2 * F * D:3 * F * D, 0]
    wc = W[3 * F * D:, :]
    wcat = jnp.concatenate([wt, wu, wp])

    item2d = item_emb.reshape(F * V, D)
    prof2d = profile_emb.reshape(F * V, D)
    ubs_t = jnp.transpose(ubs_feature, (2, 1, 0))
    tgt_f = target_ad.T.reshape(F * B)
    prof_f = profile_feature.T.reshape(F * B)

    main_fn, prof_fn = _sc_fns()
    s1 = main_fn(ubs_t, tgt_f, item2d, wcat)
    s2 = prof_fn(prof_f, prof2d, wcat)

    return _head(s1.reshape(1, B), s2.reshape(1, B), context_feature, wc,
                 b.reshape(1, 1))
